# baseline (speedup 1.0000x reference)
import jax
import jax.numpy as jnp
from jax.experimental import pallas as pl

_BR = 3200


def _linear_block(x_ref, w_ref, b_ref, o_ref):
    o_ref[...] = (
        jax.lax.dot_general(
            x_ref[...], w_ref[...],
            (((1,), (1,)), ((), ())),
            preferred_element_type=jnp.float32,
        )
        + b_ref[...]
    )


def kernel(x, edge_index, edge_attr, semantic_data, W_out, b_out):
    n, t, f, d = x.shape
    rows = n * t * f
    x2 = x.reshape(rows, d)
    b2 = b_out.reshape(1, d)
    out = pl.pallas_call(
        _linear_block,
        grid=(rows // _BR,),
        in_specs=[
            pl.BlockSpec((_BR, d), lambda i: (i, 0)),
            pl.BlockSpec((d, d), lambda i: (0, 0)),
            pl.BlockSpec((1, d), lambda i: (0, 0)),
        ],
        out_specs=pl.BlockSpec((_BR, d), lambda i: (i, 0)),
        out_shape=jax.ShapeDtypeStruct((rows, d), jnp.float32),
    )(x2, W_out, b2)
    return out.reshape(n, t, f, d)


# speedup vs baseline: 4.9780x; 4.9780x over previous
import jax
import jax.numpy as jnp
from jax.experimental import pallas as pl


def _linear_block(x_ref, w_ref, b_ref, o_ref):
    xb = x_ref[...].astype(jnp.bfloat16)
    o_ref[...] = (
        jax.lax.dot_general(
            w_ref[...], xb,
            (((1,), (0,)), ((), ())),
            preferred_element_type=jnp.float32,
        )
        + b_ref[...]
    )


def kernel(x, edge_index, edge_attr, semantic_data, W_out, b_out):
    n, t, f, d = x.shape
    k = f * d
    xt = jnp.transpose(x, (1, 2, 3, 0)).reshape(t * k, n)
    w2 = jnp.kron(jnp.eye(f, dtype=W_out.dtype), W_out).astype(jnp.bfloat16)
    b2 = jnp.tile(b_out, f).reshape(k, 1)
    out = pl.pallas_call(
        _linear_block,
        grid=(t,),
        in_specs=[
            pl.BlockSpec((k, n), lambda i: (i, 0)),
            pl.BlockSpec((k, k), lambda i: (0, 0)),
            pl.BlockSpec((k, 1), lambda i: (0, 0)),
        ],
        out_specs=pl.BlockSpec((k, n), lambda i: (i, 0)),
        out_shape=jax.ShapeDtypeStruct((t * k, n), jnp.float32),
    )(xt, w2, b2)
    return jnp.transpose(out.reshape(t, f, d, n), (3, 0, 1, 2))
